# baseline (speedup 1.0000x reference)
import functools

import numpy as np

import jax
import jax.numpy as jnp
from jax import lax
from jax.experimental import pallas as pl
from jax.experimental.pallas import tpu as pltpu
from jax.experimental.pallas import tpu_sc as plsc

N = 10000
E = 320000
LD = 64
POS = 3

NC, NS, L = 2, 16, 16
NW = NC * NS
CH = 128
K0 = 58
K1 = 102
E_PAD = NS * (K0 + K1) * CH
N_PAD = 10240
RPT = N_PAD // NS

_F32 = jnp.float32
_BF16 = jnp.bfloat16


def _edge_sc_body(S_hbm, D_hbm, src_hbm, dst_hbm, out_hbm, deg_hbm,
                  srcA, dstA, dstsA, sA, dA, hA,
                  srcB, dstB, dstsB, sB, dB, hB,
                  one_v, degbuf, G_sh, deg_sh,
                  gsemA, gsemB, ssemA, ssemB, isemA, isemB):
    c = lax.axis_index("c")
    s = lax.axis_index("s")
    zero16 = jnp.zeros((L,), _F32)

    def _zb(r, _):
        for q in range(LD // L):
            hA[r, pl.ds(q * L, L)] = zero16
        return 0
    lax.fori_loop(0, CH, _zb, 0)

    def _zdeg(i, _):
        degbuf[pl.ds(i * L, L)] = zero16
        return 0
    lax.fori_loop(0, RPT // L, _zdeg, 0)

    for q in range(CH // L):
        one_v[pl.ds(q * L, L)] = zero16 + 1.0

    for k in range(RPT // CH):
        pltpu.sync_copy(hA, G_sh.at[pl.ds(s * RPT + k * CH, CH)])
    pltpu.sync_copy(degbuf, deg_sh.at[pl.ds(s * RPT, RPT)])
    plsc.subcore_barrier()

    kcnt = jax.lax.select(c == 0, jnp.int32(K0), jnp.int32(K1))
    base = c * (NS * K0 * CH) + s * kcnt * CH
    bufs = ((srcA, dstA, dstsA, sA, dA, hA, gsemA, ssemA, isemA),
            (srcB, dstB, dstsB, sB, dB, hB, gsemB, ssemB, isemB))

    for P in range(2):
        src_v, dst_v, dsts, sb, db, hb, gsem, ssem, isem = bufs[P]
        off = base + P * CH
        pltpu.sync_copy(src_hbm.at[pl.ds(off, CH)], src_v)
        pltpu.sync_copy(dst_hbm.at[pl.ds(off, CH)], dst_v)
        pltpu.async_copy(S_hbm.at[src_v], sb, gsem)
        pltpu.async_copy(D_hbm.at[dst_v], db, gsem)

    def _pair(g, _):
        for P in range(2):
            src_v, dst_v, dsts, sb, db, hb, gsem, ssem, isem = bufs[P]
            ci = 2 * g + P
            pltpu.make_async_copy(S_hbm.at[src_v], sb, gsem).wait()
            pltpu.make_async_copy(D_hbm.at[dst_v], db, gsem).wait()

            @pl.when(g != 0)
            def _():
                pltpu.make_async_copy(hb, G_sh.at[dsts], ssem).wait()
                pltpu.make_async_copy(one_v, deg_sh.at[dsts], ssem).wait()

            for q in range(CH // L):
                dsts[pl.ds(q * L, L)] = dst_v[pl.ds(q * L, L)]

            nxt = jnp.minimum(ci + 2, kcnt - 2 + P)
            noff = base + nxt * CH
            pltpu.async_copy(src_hbm.at[pl.ds(noff, CH)], src_v, isem)
            pltpu.async_copy(dst_hbm.at[pl.ds(noff, CH)], dst_v, isem)

            @plsc.parallel_loop(0, CH, step=1, unroll=8)
            def _relu(r):
                for q in range(LD // 32):
                    v = sb[r, pl.ds(q * 32, 32)] + db[r, pl.ds(q * 32, 32)]
                    h = jnp.maximum(v, jnp.zeros((32,), _BF16))
                    ha, hbv = plsc.unpack(h, format=plsc.PackFormat.INTERLEAVED)
                    hb[r, pl.ds(q * 32, L)] = ha
                    hb[r, pl.ds(q * 32 + L, L)] = hbv

            pltpu.async_copy(hb, G_sh.at[dsts], ssem, add=True)
            pltpu.async_copy(one_v, deg_sh.at[dsts], ssem, add=True)

            pltpu.make_async_copy(src_hbm.at[pl.ds(noff, CH)], src_v, isem).wait()
            pltpu.make_async_copy(dst_hbm.at[pl.ds(noff, CH)], dst_v, isem).wait()
            pltpu.async_copy(S_hbm.at[src_v], sb, gsem)
            pltpu.async_copy(D_hbm.at[dst_v], db, gsem)
        return 0
    lax.fori_loop(0, kcnt // 2, _pair, 0)

    for P in range(2):
        src_v, dst_v, dsts, sb, db, hb, gsem, ssem, isem = bufs[P]
        pltpu.make_async_copy(S_hbm.at[src_v], sb, gsem).wait()
        pltpu.make_async_copy(D_hbm.at[dst_v], db, gsem).wait()
        pltpu.make_async_copy(hb, G_sh.at[dsts], ssem).wait()
        pltpu.make_async_copy(one_v, deg_sh.at[dsts], ssem).wait()

    plsc.subcore_barrier()
    for k in range(RPT // CH):
        pltpu.sync_copy(G_sh.at[pl.ds(s * RPT + k * CH, CH)], hA)
        pltpu.sync_copy(hA, out_hbm.at[c, pl.ds(s * RPT + k * CH, CH)])
    pltpu.sync_copy(deg_sh.at[pl.ds(s * RPT, RPT)], degbuf)
    pltpu.sync_copy(degbuf, deg_hbm.at[c, pl.ds(s * RPT, RPT)])


@functools.lru_cache(maxsize=1)
def _edge_sc_kernel():
  return pl.kernel(
    _edge_sc_body,
    out_type=(pltpu.HBM((NC, N_PAD, LD), _F32),
              pltpu.HBM((NC, N_PAD), _F32)),
    mesh=plsc.VectorSubcoreMesh(core_axis_name="c", subcore_axis_name="s",
                                num_cores=NC, num_subcores=NS),
    scratch_types=(
        pltpu.VMEM((CH,), jnp.int32),
        pltpu.VMEM((CH,), jnp.int32),
        pltpu.VMEM((CH,), jnp.int32),
        pltpu.VMEM((CH, LD), _BF16),
        pltpu.VMEM((CH, LD), _BF16),
        pltpu.VMEM((CH, LD), _F32),
        pltpu.VMEM((CH,), jnp.int32),
        pltpu.VMEM((CH,), jnp.int32),
        pltpu.VMEM((CH,), jnp.int32),
        pltpu.VMEM((CH, LD), _BF16),
        pltpu.VMEM((CH, LD), _BF16),
        pltpu.VMEM((CH, LD), _F32),
        pltpu.VMEM((CH,), _F32),
        pltpu.VMEM((RPT,), _F32),
        pltpu.VMEM_SHARED((N_PAD, LD), _F32),
        pltpu.VMEM_SHARED((N_PAD,), _F32),
        pltpu.SemaphoreType.DMA,
        pltpu.SemaphoreType.DMA,
        pltpu.SemaphoreType.DMA,
        pltpu.SemaphoreType.DMA,
        pltpu.SemaphoreType.DMA,
        pltpu.SemaphoreType.DMA,
    ),
    compiler_params=pltpu.CompilerParams(use_tc_tiling_on_sc=False,
                                         needs_layout_passes=False),
  )


def _edge_sc(S, D, srcp, dstp):
    return _edge_sc_kernel()(S, D, srcp, dstp)


BR = 2000
GRID = N // BR


def _ln_rows(h, g, b):
    m = jnp.mean(h, axis=-1, keepdims=True)
    v = jnp.mean((h - m) ** 2, axis=-1, keepdims=True)
    return (h - m) * jax.lax.rsqrt(v + 1e-5) * g + b


def _tc1_body(nin, enc_W1, enc_b1, enc_W2, enc_b2, enc_g, enc_beta,
              Ws0, Wd0, Wp0, eb1_0, Wp1,
              x_o, S_o, D_o, Pw1_o):
    z = nin[...]
    feat = z[:, 6:10]
    h = jnp.maximum(jnp.dot(feat, enc_W1[...], preferred_element_type=_F32)
                    + enc_b1[...], 0.0)
    h = jnp.dot(h, enc_W2[...], preferred_element_type=_F32) + enc_b2[...]
    x = _ln_rows(h, enc_g[...], enc_beta[...])
    wp0 = Wp0[...]
    wp1 = Wp1[...]
    pw0 = (jnp.dot(z[:, 3:6], wp0[:3], preferred_element_type=_F32)
           + jnp.dot(z[:, 0:3], wp0[3:6], preferred_element_type=_F32))
    pw1 = (jnp.dot(z[:, 3:6], wp1[:3], preferred_element_type=_F32)
           + jnp.dot(z[:, 0:3], wp1[3:6], preferred_element_type=_F32))
    x_o[...] = x
    S_o[...] = (jnp.dot(x, Ws0[...], preferred_element_type=_F32)
                - pw0).astype(_BF16)
    D_o[...] = (jnp.dot(x, Wd0[...], preferred_element_type=_F32)
                + pw0 + eb1_0[...]).astype(_BF16)
    Pw1_o[...] = pw1


def _node_update(x, G, degT, eW2l, eb2l, nW1a, nW1b, nb1l, nW2l, nb2l, gl, bl):
    H = G[0] + G[1]
    dg = degT[:, 0:1] + degT[:, 1:2]
    agg = jnp.dot(H, eW2l, preferred_element_type=_F32) + dg * eb2l
    u = jnp.maximum(jnp.dot(x, nW1a, preferred_element_type=_F32)
                    + jnp.dot(agg, nW1b, preferred_element_type=_F32)
                    + nb1l, 0.0)
    u = jnp.dot(u, nW2l, preferred_element_type=_F32) + nb2l
    return x + _ln_rows(u, gl, bl)


def _tc2_body(x, G, degT, eW2l, eb2l, nW1a, nW1b, nb1l, nW2l, nb2l, gl, bl,
              Ws1, Wd1, eb1_1, Pw1,
              x_o, S_o, D_o):
    x1 = _node_update(x[...], G[...], degT[...], eW2l[...], eb2l[...],
                      nW1a[...], nW1b[...], nb1l[...], nW2l[...], nb2l[...],
                      gl[...], bl[...])
    pw1 = Pw1[...]
    x_o[...] = x1
    S_o[...] = (jnp.dot(x1, Ws1[...], preferred_element_type=_F32)
                - pw1).astype(_BF16)
    D_o[...] = (jnp.dot(x1, Wd1[...], preferred_element_type=_F32)
                + pw1 + eb1_1[...]).astype(_BF16)


def _tc3_body(x, G, degT, eW2l, eb2l, nW1a, nW1b, nb1l, nW2l, nb2l, gl, bl,
              dec_W1, dec_b1, dec_W2p, dec_b2p,
              win8, vel8, ntar8, ntyp,
              out_o, ls_o, nz_o):
    i = pl.program_id(0)
    xf = _node_update(x[...], G[...], degT[...], eW2l[...], eb2l[...],
                      nW1a[...], nW1b[...], nb1l[...], nW2l[...], nb2l[...],
                      gl[...], bl[...])
    d = jnp.maximum(jnp.dot(xf, dec_W1[...], preferred_element_type=_F32)
                    + dec_b1[...], 0.0)
    o = jnp.dot(d, dec_W2p[...], preferred_element_type=_F32) + dec_b2p[...]
    t8 = ntar8[...]
    fix = ntyp[...] == 3.0
    p08 = jnp.where(fix, t8, win8[...])
    o = o + p08 + vel8[...]
    interior = ntyp[...] == 0.0
    out_o[...] = jnp.where(interior, o, t8)
    diff = jnp.where(interior, o - t8, 0.0)
    ls_part = jnp.sum(diff * diff)
    nz_part = 3.0 * jnp.sum(jnp.where(interior, 1.0, 0.0))

    @pl.when(i == 0)
    def _init():
        ls_o[...] = ls_part.reshape(1, 1)
        nz_o[...] = nz_part.reshape(1, 1)

    @pl.when(i != 0)
    def _acc():
        ls_o[...] = ls_o[...] + ls_part.reshape(1, 1)
        nz_o[...] = nz_o[...] + nz_part.reshape(1, 1)


def _row_spec(width):
    return pl.BlockSpec((BR, width), lambda i: (i, 0))


def _full_spec(shape):
    nd = len(shape)
    return pl.BlockSpec(shape, lambda i: (0,) * nd)


_G_spec = pl.BlockSpec((NC, BR, LD), lambda i: (0, i, 0))


def _tc1_call(nin, *ws):
    w_specs = [_full_spec(w.shape) for w in ws]
    outf = jax.ShapeDtypeStruct((N, LD), _F32)
    outb = jax.ShapeDtypeStruct((N, LD), _BF16)
    return pl.pallas_call(
        _tc1_body,
        grid=(GRID,),
        in_specs=[_row_spec(10)] + w_specs,
        out_specs=[_row_spec(LD)] * 4,
        out_shape=[outf, outb, outb, outf],
    )(nin, *ws)


def _tc2_call(x, G, degT, *ws):
    w_specs = [_full_spec(w.shape) for w in ws[:-1]] + [_row_spec(LD)]
    outf = jax.ShapeDtypeStruct((N, LD), _F32)
    outb = jax.ShapeDtypeStruct((N, LD), _BF16)
    return pl.pallas_call(
        _tc2_body,
        grid=(GRID,),
        in_specs=[_row_spec(LD), _G_spec, _row_spec(NC)] + w_specs,
        out_specs=[_row_spec(LD)] * 3,
        out_shape=[outf, outb, outb],
    )(x, G, degT, *ws)


def _tc3_call(x, G, degT, *ws):
    w_specs = ([_full_spec(w.shape) for w in ws[:-4]]
               + [_row_spec(8), _row_spec(8), _row_spec(8), _row_spec(1)])
    return pl.pallas_call(
        _tc3_body,
        grid=(GRID,),
        in_specs=[_row_spec(LD), _G_spec, _row_spec(NC)] + w_specs,
        out_specs=[_row_spec(8),
                   pl.BlockSpec((1, 1), lambda i: (0, 0)),
                   pl.BlockSpec((1, 1), lambda i: (0, 0))],
        out_shape=[jax.ShapeDtypeStruct((N, 8), _F32),
                   jax.ShapeDtypeStruct((1, 1), _F32),
                   jax.ShapeDtypeStruct((1, 1), _F32)],
    )(x, G, degT, *ws)


def kernel(node_in, node_tar, m_idx, m_gs, enc_W1, enc_b1, enc_W2, enc_b2,
           enc_g, enc_beta, eW1, eb1, eW2, eb2, nW1, nb1, nW2, nb2,
           ln_g, ln_b, dec_W1, dec_b1, dec_W2, dec_b2):
    nin = node_in[0]
    ntar = node_tar[0]
    src = m_gs[0]
    dst = m_gs[1]
    pad = E_PAD - E
    srcp = jnp.concatenate([src, jnp.zeros((pad,), src.dtype)])
    dstp = jnp.concatenate([dst, jnp.full((pad,), N, dst.dtype)])

    r1 = lambda a: a.reshape(1, -1)
    perm = np.array(
        sum(([q * 32 + 2 * j for j in range(16)]
             + [q * 32 + 2 * j + 1 for j in range(16)]
             for q in range(LD // 32)), []), np.int32)
    Ws = [eW1[l, :LD] for l in range(2)]
    Wd = [eW1[l, LD:2 * LD] for l in range(2)]
    Wp = [eW1[l, 2 * LD:] for l in range(2)]
    nW1a = [nW1[l, :LD] for l in range(2)]
    nW1b = [nW1[l, LD:] for l in range(2)]

    x, S0, D0, Pw1 = _tc1_call(
        nin, enc_W1, r1(enc_b1), enc_W2, r1(enc_b2), r1(enc_g), r1(enc_beta),
        Ws[0], Wd[0], Wp[0], r1(eb1[0]), Wp[1])

    G, deg = _edge_sc(S0, D0, srcp, dstp)
    degT = deg.T

    x1, S1, D1 = _tc2_call(
        x, G, degT, eW2[0][perm], r1(eb2[0]), nW1a[0], nW1b[0], r1(nb1[0]),
        nW2[0], r1(nb2[0]), r1(ln_g[0]), r1(ln_b[0]),
        Ws[1], Wd[1], r1(eb1[1]), Pw1)

    G2, deg2 = _edge_sc(S1, D1, srcp, dstp)
    degT2 = deg2.T

    dec_W2p = jnp.pad(dec_W2, ((0, 0), (0, 8 - POS)))
    dec_b2p = jnp.pad(r1(dec_b2), ((0, 0), (0, 8 - POS)))
    win8 = jnp.pad(nin[:, 0:POS], ((0, 0), (0, 8 - POS)))
    vel8 = jnp.pad(nin[:, 2 * POS:3 * POS], ((0, 0), (0, 8 - POS)))
    ntar8 = jnp.pad(ntar, ((0, 0), (0, 8 - POS)))
    ntyp = nin[:, 9:10]

    out8, lsum, nz = _tc3_call(
        x1, G2, degT2, eW2[1][perm], r1(eb2[1]), nW1a[1], nW1b[1], r1(nb1[1]),
        nW2[1], r1(nb2[1]), r1(ln_g[1]), r1(ln_b[1]),
        dec_W1, r1(dec_b1), dec_W2p, dec_b2p,
        win8, vel8, ntar8, ntyp)

    out = out8[:, :POS][None]
    nzs = nz[0, 0]
    return (lsum[0, 0] / nzs, out, nzs)


# speedup vs baseline: 1.1250x; 1.1250x over previous
import functools

import numpy as np

import jax
import jax.numpy as jnp
from jax import lax
from jax.experimental import pallas as pl
from jax.experimental.pallas import tpu as pltpu
from jax.experimental.pallas import tpu_sc as plsc

N = 10000
E = 320000
LD = 64
POS = 3

NC, NS, L = 2, 16, 16
NW = NC * NS
CH = 128
K0 = 102
K1 = 58
E_PAD = NS * (K0 + K1) * CH
N_PAD = 10240
RPT = N_PAD // NS

_F32 = jnp.float32
_BF16 = jnp.bfloat16


def _edge_sc_body(S_hbm, D_hbm, src_hbm, dst_hbm, out_hbm, deg_hbm,
                  srcA, dstA, dstsA, sA, dA, hA,
                  srcB, dstB, dstsB, sB, dB, hB,
                  one_v, degbuf, G_sh, deg_sh,
                  gsemA, gsemB, ssemA, ssemB, isemA, isemB):
    c = lax.axis_index("c")
    s = lax.axis_index("s")
    zero16 = jnp.zeros((L,), _F32)

    def _zb(r, _):
        for q in range(LD // L):
            hA[r, pl.ds(q * L, L)] = zero16
        return 0
    lax.fori_loop(0, CH, _zb, 0)

    def _zdeg(i, _):
        degbuf[pl.ds(i * L, L)] = zero16
        return 0
    lax.fori_loop(0, RPT // L, _zdeg, 0)

    for q in range(CH // L):
        one_v[pl.ds(q * L, L)] = zero16 + 1.0

    for k in range(RPT // CH):
        pltpu.sync_copy(hA, G_sh.at[pl.ds(s * RPT + k * CH, CH)])
    pltpu.sync_copy(degbuf, deg_sh.at[pl.ds(s * RPT, RPT)])
    plsc.subcore_barrier()

    kcnt = jax.lax.select(c == 0, jnp.int32(K0), jnp.int32(K1))
    base = c * (NS * K0 * CH) + s * kcnt * CH
    bufs = ((srcA, dstA, dstsA, sA, dA, hA, gsemA, ssemA, isemA),
            (srcB, dstB, dstsB, sB, dB, hB, gsemB, ssemB, isemB))

    for P in range(2):
        src_v, dst_v, dsts, sb, db, hb, gsem, ssem, isem = bufs[P]
        off = base + P * CH
        pltpu.sync_copy(src_hbm.at[pl.ds(off, CH)], src_v)
        pltpu.sync_copy(dst_hbm.at[pl.ds(off, CH)], dst_v)
        pltpu.async_copy(S_hbm.at[src_v], sb, gsem)
        pltpu.async_copy(D_hbm.at[dst_v], db, gsem)

    def _pair(g, _):
        for P in range(2):
            src_v, dst_v, dsts, sb, db, hb, gsem, ssem, isem = bufs[P]
            ci = 2 * g + P
            pltpu.make_async_copy(S_hbm.at[src_v], sb, gsem).wait()
            pltpu.make_async_copy(D_hbm.at[dst_v], db, gsem).wait()

            @pl.when(g != 0)
            def _():
                pltpu.make_async_copy(hb, G_sh.at[dsts], ssem).wait()
                pltpu.make_async_copy(one_v, deg_sh.at[dsts], ssem).wait()

            for q in range(CH // L):
                dsts[pl.ds(q * L, L)] = dst_v[pl.ds(q * L, L)]

            nxt = jnp.minimum(ci + 2, kcnt - 2 + P)
            noff = base + nxt * CH
            pltpu.async_copy(src_hbm.at[pl.ds(noff, CH)], src_v, isem)
            pltpu.async_copy(dst_hbm.at[pl.ds(noff, CH)], dst_v, isem)

            @plsc.parallel_loop(0, CH, step=1, unroll=8)
            def _relu(r):
                for q in range(LD // 32):
                    v = sb[r, pl.ds(q * 32, 32)] + db[r, pl.ds(q * 32, 32)]
                    h = jnp.maximum(v, jnp.zeros((32,), _BF16))
                    ha, hbv = plsc.unpack(h, format=plsc.PackFormat.INTERLEAVED)
                    hb[r, pl.ds(q * 32, L)] = ha
                    hb[r, pl.ds(q * 32 + L, L)] = hbv

            pltpu.async_copy(hb, G_sh.at[dsts], ssem, add=True)
            pltpu.async_copy(one_v, deg_sh.at[dsts], ssem, add=True)

            pltpu.make_async_copy(src_hbm.at[pl.ds(noff, CH)], src_v, isem).wait()
            pltpu.make_async_copy(dst_hbm.at[pl.ds(noff, CH)], dst_v, isem).wait()
            pltpu.async_copy(S_hbm.at[src_v], sb, gsem)
            pltpu.async_copy(D_hbm.at[dst_v], db, gsem)
        return 0
    lax.fori_loop(0, kcnt // 2, _pair, 0)

    for P in range(2):
        src_v, dst_v, dsts, sb, db, hb, gsem, ssem, isem = bufs[P]
        pltpu.make_async_copy(S_hbm.at[src_v], sb, gsem).wait()
        pltpu.make_async_copy(D_hbm.at[dst_v], db, gsem).wait()
        pltpu.make_async_copy(hb, G_sh.at[dsts], ssem).wait()
        pltpu.make_async_copy(one_v, deg_sh.at[dsts], ssem).wait()

    plsc.subcore_barrier()
    for k in range(RPT // CH):
        pltpu.sync_copy(G_sh.at[pl.ds(s * RPT + k * CH, CH)], hA)
        pltpu.sync_copy(hA, out_hbm.at[c, pl.ds(s * RPT + k * CH, CH)])
    pltpu.sync_copy(deg_sh.at[pl.ds(s * RPT, RPT)], degbuf)
    pltpu.sync_copy(degbuf, deg_hbm.at[c, pl.ds(s * RPT, RPT)])


@functools.lru_cache(maxsize=1)
def _edge_sc_kernel():
  return pl.kernel(
    _edge_sc_body,
    out_type=(pltpu.HBM((NC, N_PAD, LD), _F32),
              pltpu.HBM((NC, N_PAD), _F32)),
    mesh=plsc.VectorSubcoreMesh(core_axis_name="c", subcore_axis_name="s",
                                num_cores=NC, num_subcores=NS),
    scratch_types=(
        pltpu.VMEM((CH,), jnp.int32),
        pltpu.VMEM((CH,), jnp.int32),
        pltpu.VMEM((CH,), jnp.int32),
        pltpu.VMEM((CH, LD), _BF16),
        pltpu.VMEM((CH, LD), _BF16),
        pltpu.VMEM((CH, LD), _F32),
        pltpu.VMEM((CH,), jnp.int32),
        pltpu.VMEM((CH,), jnp.int32),
        pltpu.VMEM((CH,), jnp.int32),
        pltpu.VMEM((CH, LD), _BF16),
        pltpu.VMEM((CH, LD), _BF16),
        pltpu.VMEM((CH, LD), _F32),
        pltpu.VMEM((CH,), _F32),
        pltpu.VMEM((RPT,), _F32),
        pltpu.VMEM_SHARED((N_PAD, LD), _F32),
        pltpu.VMEM_SHARED((N_PAD,), _F32),
        pltpu.SemaphoreType.DMA,
        pltpu.SemaphoreType.DMA,
        pltpu.SemaphoreType.DMA,
        pltpu.SemaphoreType.DMA,
        pltpu.SemaphoreType.DMA,
        pltpu.SemaphoreType.DMA,
    ),
    compiler_params=pltpu.CompilerParams(use_tc_tiling_on_sc=False,
                                         needs_layout_passes=False),
  )


def _edge_sc(S, D, srcp, dstp):
    return _edge_sc_kernel()(S, D, srcp, dstp)


BR = 2000
GRID = N // BR


def _ln_rows(h, g, b):
    m = jnp.mean(h, axis=-1, keepdims=True)
    v = jnp.mean((h - m) ** 2, axis=-1, keepdims=True)
    return (h - m) * jax.lax.rsqrt(v + 1e-5) * g + b


def _tc1_body(nin, enc_W1, enc_b1, enc_W2, enc_b2, enc_g, enc_beta,
              Ws0, Wd0, Wp0, eb1_0, Wp1,
              x_o, S_o, D_o, Pw1_o):
    z = nin[...]
    feat = z[:, 6:10]
    h = jnp.maximum(jnp.dot(feat, enc_W1[...], preferred_element_type=_F32)
                    + enc_b1[...], 0.0)
    h = jnp.dot(h, enc_W2[...], preferred_element_type=_F32) + enc_b2[...]
    x = _ln_rows(h, enc_g[...], enc_beta[...])
    wp0 = Wp0[...]
    wp1 = Wp1[...]
    pw0 = (jnp.dot(z[:, 3:6], wp0[:3], preferred_element_type=_F32)
           + jnp.dot(z[:, 0:3], wp0[3:6], preferred_element_type=_F32))
    pw1 = (jnp.dot(z[:, 3:6], wp1[:3], preferred_element_type=_F32)
           + jnp.dot(z[:, 0:3], wp1[3:6], preferred_element_type=_F32))
    x_o[...] = x
    S_o[...] = (jnp.dot(x, Ws0[...], preferred_element_type=_F32)
                - pw0).astype(_BF16)
    D_o[...] = (jnp.dot(x, Wd0[...], preferred_element_type=_F32)
                + pw0 + eb1_0[...]).astype(_BF16)
    Pw1_o[...] = pw1


def _node_update(x, G, degT, eW2l, eb2l, nW1a, nW1b, nb1l, nW2l, nb2l, gl, bl):
    H = G[0] + G[1]
    dg = degT[:, 0:1] + degT[:, 1:2]
    agg = jnp.dot(H, eW2l, preferred_element_type=_F32) + dg * eb2l
    u = jnp.maximum(jnp.dot(x, nW1a, preferred_element_type=_F32)
                    + jnp.dot(agg, nW1b, preferred_element_type=_F32)
                    + nb1l, 0.0)
    u = jnp.dot(u, nW2l, preferred_element_type=_F32) + nb2l
    return x + _ln_rows(u, gl, bl)


def _tc2_body(x, G, degT, eW2l, eb2l, nW1a, nW1b, nb1l, nW2l, nb2l, gl, bl,
              Ws1, Wd1, eb1_1, Pw1,
              x_o, S_o, D_o):
    x1 = _node_update(x[...], G[...], degT[...], eW2l[...], eb2l[...],
                      nW1a[...], nW1b[...], nb1l[...], nW2l[...], nb2l[...],
                      gl[...], bl[...])
    pw1 = Pw1[...]
    x_o[...] = x1
    S_o[...] = (jnp.dot(x1, Ws1[...], preferred_element_type=_F32)
                - pw1).astype(_BF16)
    D_o[...] = (jnp.dot(x1, Wd1[...], preferred_element_type=_F32)
                + pw1 + eb1_1[...]).astype(_BF16)


def _tc3_body(x, G, degT, eW2l, eb2l, nW1a, nW1b, nb1l, nW2l, nb2l, gl, bl,
              dec_W1, dec_b1, dec_W2p, dec_b2p,
              win8, vel8, ntar8, ntyp,
              out_o, ls_o, nz_o):
    i = pl.program_id(0)
    xf = _node_update(x[...], G[...], degT[...], eW2l[...], eb2l[...],
                      nW1a[...], nW1b[...], nb1l[...], nW2l[...], nb2l[...],
                      gl[...], bl[...])
    d = jnp.maximum(jnp.dot(xf, dec_W1[...], preferred_element_type=_F32)
                    + dec_b1[...], 0.0)
    o = jnp.dot(d, dec_W2p[...], preferred_element_type=_F32) + dec_b2p[...]
    t8 = ntar8[...]
    fix = ntyp[...] == 3.0
    p08 = jnp.where(fix, t8, win8[...])
    o = o + p08 + vel8[...]
    interior = ntyp[...] == 0.0
    out_o[...] = jnp.where(interior, o, t8)
    diff = jnp.where(interior, o - t8, 0.0)
    ls_part = jnp.sum(diff * diff)
    nz_part = 3.0 * jnp.sum(jnp.where(interior, 1.0, 0.0))

    @pl.when(i == 0)
    def _init():
        ls_o[...] = ls_part.reshape(1, 1)
        nz_o[...] = nz_part.reshape(1, 1)

    @pl.when(i != 0)
    def _acc():
        ls_o[...] = ls_o[...] + ls_part.reshape(1, 1)
        nz_o[...] = nz_o[...] + nz_part.reshape(1, 1)


def _row_spec(width):
    return pl.BlockSpec((BR, width), lambda i: (i, 0))


def _full_spec(shape):
    nd = len(shape)
    return pl.BlockSpec(shape, lambda i: (0,) * nd)


_G_spec = pl.BlockSpec((NC, BR, LD), lambda i: (0, i, 0))


def _tc1_call(nin, *ws):
    w_specs = [_full_spec(w.shape) for w in ws]
    outf = jax.ShapeDtypeStruct((N, LD), _F32)
    outb = jax.ShapeDtypeStruct((N, LD), _BF16)
    return pl.pallas_call(
        _tc1_body,
        grid=(GRID,),
        in_specs=[_row_spec(10)] + w_specs,
        out_specs=[_row_spec(LD)] * 4,
        out_shape=[outf, outb, outb, outf],
    )(nin, *ws)


def _tc2_call(x, G, degT, *ws):
    w_specs = [_full_spec(w.shape) for w in ws[:-1]] + [_row_spec(LD)]
    outf = jax.ShapeDtypeStruct((N, LD), _F32)
    outb = jax.ShapeDtypeStruct((N, LD), _BF16)
    return pl.pallas_call(
        _tc2_body,
        grid=(GRID,),
        in_specs=[_row_spec(LD), _G_spec, _row_spec(NC)] + w_specs,
        out_specs=[_row_spec(LD)] * 3,
        out_shape=[outf, outb, outb],
    )(x, G, degT, *ws)


def _tc3_call(x, G, degT, *ws):
    w_specs = ([_full_spec(w.shape) for w in ws[:-4]]
               + [_row_spec(8), _row_spec(8), _row_spec(8), _row_spec(1)])
    return pl.pallas_call(
        _tc3_body,
        grid=(GRID,),
        in_specs=[_row_spec(LD), _G_spec, _row_spec(NC)] + w_specs,
        out_specs=[_row_spec(8),
                   pl.BlockSpec((1, 1), lambda i: (0, 0)),
                   pl.BlockSpec((1, 1), lambda i: (0, 0))],
        out_shape=[jax.ShapeDtypeStruct((N, 8), _F32),
                   jax.ShapeDtypeStruct((1, 1), _F32),
                   jax.ShapeDtypeStruct((1, 1), _F32)],
    )(x, G, degT, *ws)


def kernel(node_in, node_tar, m_idx, m_gs, enc_W1, enc_b1, enc_W2, enc_b2,
           enc_g, enc_beta, eW1, eb1, eW2, eb2, nW1, nb1, nW2, nb2,
           ln_g, ln_b, dec_W1, dec_b1, dec_W2, dec_b2):
    nin = node_in[0]
    ntar = node_tar[0]
    src = m_gs[0]
    dst = m_gs[1]
    pad = E_PAD - E
    srcp = jnp.concatenate([src, jnp.zeros((pad,), src.dtype)])
    dstp = jnp.concatenate([dst, jnp.full((pad,), N, dst.dtype)])

    r1 = lambda a: a.reshape(1, -1)
    perm = np.array(
        sum(([q * 32 + 2 * j for j in range(16)]
             + [q * 32 + 2 * j + 1 for j in range(16)]
             for q in range(LD // 32)), []), np.int32)
    Ws = [eW1[l, :LD] for l in range(2)]
    Wd = [eW1[l, LD:2 * LD] for l in range(2)]
    Wp = [eW1[l, 2 * LD:] for l in range(2)]
    nW1a = [nW1[l, :LD] for l in range(2)]
    nW1b = [nW1[l, LD:] for l in range(2)]

    x, S0, D0, Pw1 = _tc1_call(
        nin, enc_W1, r1(enc_b1), enc_W2, r1(enc_b2), r1(enc_g), r1(enc_beta),
        Ws[0], Wd[0], Wp[0], r1(eb1[0]), Wp[1])

    G, deg = _edge_sc(S0, D0, srcp, dstp)
    degT = deg.T

    x1, S1, D1 = _tc2_call(
        x, G, degT, eW2[0][perm], r1(eb2[0]), nW1a[0], nW1b[0], r1(nb1[0]),
        nW2[0], r1(nb2[0]), r1(ln_g[0]), r1(ln_b[0]),
        Ws[1], Wd[1], r1(eb1[1]), Pw1)

    G2, deg2 = _edge_sc(S1, D1, srcp, dstp)
    degT2 = deg2.T

    dec_W2p = jnp.pad(dec_W2, ((0, 0), (0, 8 - POS)))
    dec_b2p = jnp.pad(r1(dec_b2), ((0, 0), (0, 8 - POS)))
    win8 = jnp.pad(nin[:, 0:POS], ((0, 0), (0, 8 - POS)))
    vel8 = jnp.pad(nin[:, 2 * POS:3 * POS], ((0, 0), (0, 8 - POS)))
    ntar8 = jnp.pad(ntar, ((0, 0), (0, 8 - POS)))
    ntyp = nin[:, 9:10]

    out8, lsum, nz = _tc3_call(
        x1, G2, degT2, eW2[1][perm], r1(eb2[1]), nW1a[1], nW1b[1], r1(nb1[1]),
        nW2[1], r1(nb2[1]), r1(ln_g[1]), r1(ln_b[1]),
        dec_W1, r1(dec_b1), dec_W2p, dec_b2p,
        win8, vel8, ntar8, ntyp)

    out = out8[:, :POS][None]
    nzs = nz[0, 0]
    return (lsum[0, 0] / nzs, out, nzs)
